# baseline (speedup 1.0000x reference)
import functools

import jax
import jax.numpy as jnp
from jax import lax
from jax.experimental import pallas as pl
from jax.experimental.pallas import tpu as pltpu
from jax.experimental.pallas import tpu_sc as plsc

_EMBED = 64
_HIDDEN = 128
_NC, _NS = 2, 16
_NW = _NC * _NS
_CHUNK = 256


def _sc_gather2(user_table2, user_idx, movie_table2, movie_idx):
    b = user_idx.shape[0]
    b_per_w = b // _NW
    nchunks = b_per_w // _CHUNK
    mesh = plsc.VectorSubcoreMesh(core_axis_name="c", subcore_axis_name="s")

    @functools.partial(
        pl.kernel,
        out_type=[
            jax.ShapeDtypeStruct((b, 2 * _EMBED), jnp.float32),
            jax.ShapeDtypeStruct((b, 2 * _EMBED), jnp.float32),
        ],
        mesh=mesh,
        scratch_types=[
            pltpu.VMEM((_CHUNK,), jnp.int32),
            pltpu.VMEM((_CHUNK, 2 * _EMBED), jnp.float32),
            pltpu.VMEM((_CHUNK,), jnp.int32),
            pltpu.VMEM((_CHUNK, 2 * _EMBED), jnp.float32),
            pltpu.SemaphoreType.DMA,
            pltpu.SemaphoreType.DMA,
        ],
        compiler_params=pltpu.CompilerParams(use_tc_tiling_on_sc=True),
    )
    def k(ut_hbm, ui_hbm, mt_hbm, mi_hbm, uo_hbm, mo_hbm,
          ui_v, ur_v, mi_v, mr_v, usem, msem):
        wid = lax.axis_index("s") * _NC + lax.axis_index("c")
        base = wid * b_per_w

        @pl.loop(0, nchunks)
        def _(ci):
            off = base + ci * _CHUNK
            pltpu.sync_copy(ui_hbm.at[pl.ds(off, _CHUNK)], ui_v)
            pltpu.sync_copy(mi_hbm.at[pl.ds(off, _CHUNK)], mi_v)
            cu = pltpu.async_copy(ut_hbm.at[ui_v], ur_v, usem)
            cm = pltpu.async_copy(mt_hbm.at[mi_v], mr_v, msem)
            cu.wait()
            pltpu.sync_copy(ur_v, uo_hbm.at[pl.ds(off, _CHUNK)])
            cm.wait()
            pltpu.sync_copy(mr_v, mo_hbm.at[pl.ds(off, _CHUNK)])

    return k(user_table2, user_idx, movie_table2, movie_idx)


def _mlp_body(u_ref, m_ref, pu_ref, pm_ref, w1u_ref, w1m_ref, b1_ref, w2_ref,
              b2_ref, o_ref):
    blk = u_ref.shape[0]
    lane = lax.broadcasted_iota(jnp.int32, (blk, 2 * _EMBED), 1)
    hi = lane >= _EMBED
    u_x = jnp.where(hi == (pu_ref[...] != 0), u_ref[...], 0.0)
    m_x = jnp.where(hi == (pm_ref[...] != 0), m_ref[...], 0.0)
    h = (
        jnp.dot(u_x, w1u_ref[...], preferred_element_type=jnp.float32)
        + jnp.dot(m_x, w1m_ref[...], preferred_element_type=jnp.float32)
        + b1_ref[...]
    )
    h = jnp.maximum(h, 0.0)
    o_ref[...] = (
        jnp.dot(h, w2_ref[...], preferred_element_type=jnp.float32)
        + b2_ref[0, 0]
    )


def _tc_mlp(u_rows, m_rows, pu, pm, W1, b1, W2, b2):
    b = u_rows.shape[0]
    blk = 2048
    w1u_t = W1[:, :_EMBED].T
    w1m_t = W1[:, _EMBED:].T
    w1u2 = jnp.concatenate([w1u_t, w1u_t], axis=0)
    w1m2 = jnp.concatenate([w1m_t, w1m_t], axis=0)
    out = pl.pallas_call(
        _mlp_body,
        grid=(b // blk,),
        in_specs=[
            pl.BlockSpec((blk, 2 * _EMBED), lambda i: (i, 0)),
            pl.BlockSpec((blk, 2 * _EMBED), lambda i: (i, 0)),
            pl.BlockSpec((blk, 1), lambda i: (i, 0)),
            pl.BlockSpec((blk, 1), lambda i: (i, 0)),
            pl.BlockSpec((2 * _EMBED, _HIDDEN), lambda i: (0, 0)),
            pl.BlockSpec((2 * _EMBED, _HIDDEN), lambda i: (0, 0)),
            pl.BlockSpec((1, _HIDDEN), lambda i: (0, 0)),
            pl.BlockSpec((_HIDDEN, 1), lambda i: (0, 0)),
            pl.BlockSpec((1, 1), lambda i: (0, 0)),
        ],
        out_specs=pl.BlockSpec((blk, 1), lambda i: (i, 0)),
        out_shape=jax.ShapeDtypeStruct((b, 1), jnp.float32),
    )(u_rows, m_rows, pu, pm, w1u2, w1m2, b1.reshape(1, _HIDDEN),
      W2.reshape(_HIDDEN, 1), b2.reshape(1, 1))
    return out.reshape(b)


def kernel(user_input, movie_input, user_table, movie_table, W1, b1, W2, b2):
    b = user_input.shape[0]
    ui = user_input.astype(jnp.int32)
    mi = movie_input.astype(jnp.int32)
    ut2 = user_table.reshape(user_table.shape[0] // 2, 2 * _EMBED)
    mt2 = movie_table.reshape(movie_table.shape[0] // 2, 2 * _EMBED)
    u_rows, m_rows = _sc_gather2(ut2, ui >> 1, mt2, mi >> 1)
    pu = (ui & 1).reshape(b, 1)
    pm = (mi & 1).reshape(b, 1)
    return _tc_mlp(u_rows, m_rows, pu, pm, W1, b1, W2, b2)


# speedup vs baseline: 1.2064x; 1.2064x over previous
import functools

import jax
import jax.numpy as jnp
from jax import lax
from jax.experimental import pallas as pl
from jax.experimental.pallas import tpu as pltpu
from jax.experimental.pallas import tpu_sc as plsc

_EMBED = 64
_HIDDEN = 128
_NC, _NS = 2, 16
_NW = _NC * _NS
_RELAY_C = 2048


def _fold_body(x_ref, o_ref):
    t = jnp.transpose(x_ref[...])
    o_ref[...] = jnp.concatenate(
        [t[: _RELAY_C // 2], t[_RELAY_C // 2:]], axis=1)


def _tc_fold_transpose(tbl_t):
    n = tbl_t.shape[1]
    nblk = pl.cdiv(n, _RELAY_C)
    return pl.pallas_call(
        _fold_body,
        grid=(nblk,),
        in_specs=[pl.BlockSpec((_EMBED, _RELAY_C), lambda k: (0, k))],
        out_specs=pl.BlockSpec((_RELAY_C // 2, 2 * _EMBED), lambda k: (k, 0)),
        out_shape=jax.ShapeDtypeStruct(
            (nblk * (_RELAY_C // 2), 2 * _EMBED), jnp.float32),
    )(tbl_t)


def _sc_gather1(tbl2, idx):
    b = idx.shape[0]
    bw = b // _NW
    mesh = plsc.VectorSubcoreMesh(core_axis_name="c", subcore_axis_name="s")

    @functools.partial(
        pl.kernel,
        out_type=jax.ShapeDtypeStruct((b, 2 * _EMBED), jnp.float32),
        mesh=mesh,
        scratch_types=[
            pltpu.VMEM((bw,), jnp.int32),
            pltpu.VMEM((bw, 2 * _EMBED), jnp.float32),
            pltpu.SemaphoreType.DMA,
        ],
        compiler_params=pltpu.CompilerParams(use_tc_tiling_on_sc=True),
    )
    def k(t_hbm, i_hbm, o_hbm, i_v, r_v, sem):
        wid = lax.axis_index("s") * _NC + lax.axis_index("c")
        base = wid * bw
        pltpu.sync_copy(i_hbm.at[pl.ds(base, bw)], i_v)
        pltpu.async_copy(t_hbm.at[i_v], r_v, sem).wait()
        pltpu.sync_copy(r_v, o_hbm.at[pl.ds(base, bw)])

    return k(tbl2, idx)


def _mlp_body(u_ref, m_ref, pu_ref, pm_ref, w1u_ref, w1m_ref, b1_ref, w2_ref,
              b2_ref, o_ref):
    blk = u_ref.shape[0]
    lane = lax.broadcasted_iota(jnp.int32, (blk, 2 * _EMBED), 1)
    hi = lane >= _EMBED
    u_x = jnp.where(hi == (pu_ref[...] != 0), u_ref[...], 0.0)
    m_x = jnp.where(hi == (pm_ref[...] != 0), m_ref[...], 0.0)
    h = (
        jnp.dot(u_x, w1u_ref[...], preferred_element_type=jnp.float32)
        + jnp.dot(m_x, w1m_ref[...], preferred_element_type=jnp.float32)
        + b1_ref[...]
    )
    h = jnp.maximum(h, 0.0)
    o_ref[...] = (
        jnp.dot(h, w2_ref[...], preferred_element_type=jnp.float32)
        + b2_ref[0, 0]
    )


def _tc_mlp(u_rows, m_rows, pu, pm, W1, b1, W2, b2):
    b = u_rows.shape[0]
    blk = 2048
    w1u_t = W1[:, :_EMBED].T
    w1m_t = W1[:, _EMBED:].T
    w1u2 = jnp.concatenate([w1u_t, w1u_t], axis=0)
    w1m2 = jnp.concatenate([w1m_t, w1m_t], axis=0)
    out = pl.pallas_call(
        _mlp_body,
        grid=(b // blk,),
        in_specs=[
            pl.BlockSpec((blk, 2 * _EMBED), lambda i: (i, 0)),
            pl.BlockSpec((blk, 2 * _EMBED), lambda i: (i, 0)),
            pl.BlockSpec((blk, 1), lambda i: (i, 0)),
            pl.BlockSpec((blk, 1), lambda i: (i, 0)),
            pl.BlockSpec((2 * _EMBED, _HIDDEN), lambda i: (0, 0)),
            pl.BlockSpec((2 * _EMBED, _HIDDEN), lambda i: (0, 0)),
            pl.BlockSpec((1, _HIDDEN), lambda i: (0, 0)),
            pl.BlockSpec((_HIDDEN, 1), lambda i: (0, 0)),
            pl.BlockSpec((1, 1), lambda i: (0, 0)),
        ],
        out_specs=pl.BlockSpec((blk, 1), lambda i: (i, 0)),
        out_shape=jax.ShapeDtypeStruct((b, 1), jnp.float32),
    )(u_rows, m_rows, pu, pm, w1u2, w1m2, b1.reshape(1, _HIDDEN),
      W2.reshape(_HIDDEN, 1), b2.reshape(1, 1))
    return out.reshape(b)


def _fold_index(i):
    half = _RELAY_C // 2
    gi = (i // _RELAY_C) * half + (i % half)
    parity = (i % _RELAY_C) // half
    return gi, parity


def kernel(user_input, movie_input, user_table, movie_table, W1, b1, W2, b2):
    b = user_input.shape[0]
    ui = user_input.astype(jnp.int32)
    mi = movie_input.astype(jnp.int32)
    gu, pu = _fold_index(ui)
    gm, pm = _fold_index(mi)
    ut2 = _tc_fold_transpose(user_table.T)
    u_rows = _sc_gather1(ut2, gu)
    mt2 = _tc_fold_transpose(movie_table.T)
    m_rows = _sc_gather1(mt2, gm)
    return _tc_mlp(u_rows, m_rows, pu.reshape(b, 1), pm.reshape(b, 1),
                   W1, b1, W2, b2)
